# baseline (speedup 1.0000x reference)
import jax
import jax.numpy as jnp
from jax import lax
from jax.experimental import pallas as pl
from jax.experimental.pallas import tpu as pltpu
from jax.experimental.pallas import tpu_sc as plsc

_TOPK = 3
_MARGIN = 100.0
_LAMBDA_SPARSITY = 0.008
_LAMBDA_SMOOTH = 0.0008
_HARD_NEG_RATIO = 0.3

_I32_MIN = -2147483648
_I32_MAGN = 2147483647

_NC = 2
_NS = 16
_LANES = 16

_B = 128
_T = 8192
_ROWS_PER_W = _B // (_NC * _NS)
_VREGS = _T // _LANES
_UNROLL = 8


def _row_top3_sum(row_ref):
    neg_inf = jnp.full((_LANES,), -jnp.inf, jnp.float32)

    def body(j, carry):
        m1, m2, m3 = carry
        for u in range(_UNROLL):
            v = row_ref[pl.ds((j * _UNROLL + u) * _LANES, _LANES)]
            a = jnp.minimum(m1, v)
            m1 = jnp.maximum(m1, v)
            b = jnp.minimum(m2, a)
            m2 = jnp.maximum(m2, a)
            m3 = jnp.maximum(m3, b)
        return m1, m2, m3

    m1, m2, m3 = lax.fori_loop(0, _VREGS // _UNROLL, body,
                               (neg_inf, neg_inf, neg_inf))
    t1 = jnp.float32(-jnp.inf)
    t2 = jnp.float32(-jnp.inf)
    t3 = jnp.float32(-jnp.inf)
    for vec in (m1, m2, m3):
        for l in range(_LANES):
            v = vec[l]
            a = jnp.minimum(t1, v)
            t1 = jnp.maximum(t1, v)
            b = jnp.minimum(t2, a)
            t2 = jnp.maximum(t2, a)
            t3 = jnp.maximum(t3, b)
    return t1 + t2 + t3


def _sc_top3_body(anom_hbm, out_hbm, row_v, vals_v):
    wid = lax.axis_index("s") * _NC + lax.axis_index("c")
    lane = lax.iota(jnp.int32, _LANES)

    vals = jnp.zeros((_LANES,), jnp.float32)
    for i in range(_ROWS_PER_W):
        r = wid * _ROWS_PER_W + i
        pltpu.sync_copy(anom_hbm.at[r], row_v)
        s3 = _row_top3_sum(row_v)
        vals = jnp.where(lane == i, s3, vals)

    vals_v[...] = vals
    pltpu.sync_copy(vals_v, out_hbm.at[wid])


def _sc_top3(anom):
    mesh = plsc.VectorSubcoreMesh(core_axis_name="c", subcore_axis_name="s",
                                  num_cores=_NC, num_subcores=_NS)
    fn = pl.kernel(
        _sc_top3_body,
        out_type=jax.ShapeDtypeStruct((_NC * _NS, _LANES), jnp.float32),
        mesh=mesh,
        scratch_types=[
            pltpu.VMEM((_T,), jnp.float32),
            pltpu.VMEM((_LANES,), jnp.float32),
        ],
    )
    return fn(anom)


def _unkey(s):
    i = s ^ (lax.shift_right_arithmetic(s, 31) & jnp.int32(_I32_MAGN))
    return lax.bitcast_convert_type(i, jnp.float32)


def _topk_mean2(x, k):
    rows = x.shape[0]
    kf = jnp.float32(k)
    b = lax.bitcast_convert_type(x, jnp.int32)
    key = b ^ (lax.shift_right_arithmetic(b, 31) & jnp.int32(_I32_MAGN))
    uk = key ^ jnp.int32(_I32_MIN)
    hi16 = (lax.shift_right_logical(uk, 16) - 32768).astype(jnp.int16)
    lo16 = ((uk & jnp.int32(0xFFFF)) - 32768).astype(jnp.int16)

    def body_a(it, p):
        c = p | lax.shift_left(jnp.int32(1), 15 - it)
        thr = (c - 32768).astype(jnp.int16)
        cnt = jnp.sum(jnp.where(hi16 >= thr, jnp.int16(1), jnp.int16(0)),
                      axis=1, keepdims=True, dtype=jnp.int32)
        return jnp.where(cnt >= k, c, p)

    pa = lax.fori_loop(0, 16, body_a, jnp.zeros((rows, 1), jnp.int32))
    pa16 = (pa - 32768).astype(jnp.int16)
    n_hi = jnp.sum(jnp.where(hi16 > pa16, jnp.int16(1), jnp.int16(0)),
                   axis=1, keepdims=True, dtype=jnp.int32)
    ties = hi16 == pa16
    r = k - n_hi

    def body_b(it, p):
        c = p | lax.shift_left(jnp.int32(1), 15 - it)
        thr = (c - 32768).astype(jnp.int16)
        cnt = jnp.sum(
            jnp.where(ties & (lo16 >= thr), jnp.int16(1), jnp.int16(0)),
            axis=1, keepdims=True, dtype=jnp.int32)
        return jnp.where(cnt >= r, c, p)

    pb = lax.fori_loop(0, 16, body_b, jnp.zeros((rows, 1), jnp.int32))
    tf = _unkey((lax.shift_left(pa, 16) | pb) ^ jnp.int32(_I32_MIN))
    gt = x > tf
    n_gt = jnp.sum(jnp.where(gt, 1.0, 0.0), axis=1, keepdims=True)
    sum_gt = jnp.sum(jnp.where(gt, x, 0.0), axis=1, keepdims=True)
    return (sum_gt + (kf - n_gt) * tf) / kf


def _tc_main_body(anom_ref, norm_ref, nmean_ref, sp_ref, sm_ref):
    anom = anom_ref[...]
    norm = norm_ref[...]
    b_a, t_a = anom.shape
    b_n, t_n = norm.shape
    hard_k = max(1, int(t_n * _HARD_NEG_RATIO))

    nmean_ref[...] = _topk_mean2(norm, hard_k)

    sp_ref[0, 0] = (jnp.sum(anom) / jnp.float32(b_a * t_a)
                    + jnp.sum(norm) / jnp.float32(b_n * t_n)) * 0.5

    diff_a = anom[:, 1:] - anom[:, :-1]
    diff_n = norm[:, 1:] - norm[:, :-1]
    sm_ref[0, 0] = (jnp.sum(diff_a * diff_a) / jnp.float32(b_a * (t_a - 1))
                    + jnp.sum(diff_n * diff_n)
                    / jnp.float32(b_n * (t_n - 1))) * 0.5


def _tc_main(anom, norm):
    scalar = jax.ShapeDtypeStruct((1, 1), jnp.float32)
    smem = pl.BlockSpec(memory_space=pltpu.SMEM)
    return pl.pallas_call(
        _tc_main_body,
        out_shape=(jax.ShapeDtypeStruct((_B, 1), jnp.float32), scalar,
                   scalar),
        in_specs=[pl.BlockSpec(memory_space=pltpu.VMEM)] * 2,
        out_specs=(pl.BlockSpec(memory_space=pltpu.VMEM), smem, smem),
    )(anom, norm)


def _tc_final_body(asums_ref, nmean_ref, sp_ref, sm_ref,
                   total_ref, rank_ref, osp_ref, osm_ref):
    asums = asums_ref[...]
    n_mean = nmean_ref[...]

    nw = _NC * _NS
    i_row = lax.broadcasted_iota(jnp.int32, (_B, nw), 0)
    w_col = lax.broadcasted_iota(jnp.int32, (_B, nw), 1)
    sel = jnp.where(w_col == lax.shift_right_logical(i_row, 2), 1.0, 0.0)
    b1 = lax.dot_general(sel, asums,
                         dimension_numbers=(((1,), (0,)), ((), ())),
                         preferred_element_type=jnp.float32)
    i_row2 = lax.broadcasted_iota(jnp.int32, (_B, _LANES), 0)
    c_col = lax.broadcasted_iota(jnp.int32, (_B, _LANES), 1)
    pick = jnp.where(c_col == (i_row2 & 3), 1.0, 0.0)
    a_sum3 = jnp.sum(b1 * pick, axis=1, keepdims=True)
    a_mean = a_sum3 / jnp.float32(_TOPK)

    ones_col = jnp.ones((_B, 1), jnp.float32)
    norm_row = lax.dot_general(
        ones_col, n_mean,
        dimension_numbers=(((1,), (1,)), ((), ())),
        preferred_element_type=jnp.float32,
    )
    pairs = jnp.maximum(_MARGIN - a_mean + norm_row, 0.0)
    rank_loss = jnp.sum(pairs) / jnp.float32(_B * _B)

    sparsity = sp_ref[0, 0]
    smooth = sm_ref[0, 0]
    total_ref[0, 0] = rank_loss + _LAMBDA_SPARSITY * sparsity \
        + _LAMBDA_SMOOTH * smooth
    rank_ref[0, 0] = rank_loss
    osp_ref[0, 0] = sparsity
    osm_ref[0, 0] = smooth


def _tc_final(a_sums, n_mean, sp, sm):
    scalar = jax.ShapeDtypeStruct((1, 1), jnp.float32)
    smem = pl.BlockSpec(memory_space=pltpu.SMEM)
    vmem = pl.BlockSpec(memory_space=pltpu.VMEM)
    return pl.pallas_call(
        _tc_final_body,
        out_shape=(scalar, scalar, scalar, scalar),
        in_specs=(vmem, vmem, smem, smem),
        out_specs=(smem, smem, smem, smem),
    )(a_sums, n_mean, sp, sm)


def kernel(anom_scores, norm_scores):
    a_sums = _sc_top3(anom_scores)
    n_mean, sp, sm = _tc_main(anom_scores, norm_scores)
    total, rank, osp, osm = _tc_final(a_sums, n_mean, sp, sm)
    return (total[0, 0], rank[0, 0], osp[0, 0], osm[0, 0])


# speedup vs baseline: 1.2863x; 1.2863x over previous
import jax
import jax.numpy as jnp
from jax import lax
from jax.experimental import pallas as pl
from jax.experimental.pallas import tpu as pltpu
from jax.experimental.pallas import tpu_sc as plsc

_TOPK = 3
_MARGIN = 100.0
_LAMBDA_SPARSITY = 0.008
_LAMBDA_SMOOTH = 0.0008
_HARD_NEG_RATIO = 0.3

_I32_MIN = -2147483648
_I32_MAGN = 2147483647

_NC = 2
_NS = 16
_LANES = 16

_B = 128
_T = 8192
_ROWS_PER_W = _B // (_NC * _NS)
_VREGS = _T // _LANES
_UNROLL = 8


def _lane_total(vec):
    s = vec[0]
    for l in range(1, _LANES):
        s = s + vec[l]
    return s


def _row_stats(row_ref, with_top3):
    neg_inf = jnp.full((_LANES,), -jnp.inf, jnp.float32)
    zero = jnp.zeros((_LANES,), jnp.float32)
    n_full = (_VREGS - 1) // _UNROLL

    def step(v, vn, carry):
        m1, m2, m3, acc, d2 = carry
        acc = acc + v
        d = vn - v
        d2 = d2 + d * d
        if with_top3:
            a = jnp.minimum(m1, v)
            m1 = jnp.maximum(m1, v)
            b = jnp.minimum(m2, a)
            m2 = jnp.maximum(m2, a)
            m3 = jnp.maximum(m3, b)
        return m1, m2, m3, acc, d2

    def body(j, carry):
        for u in range(_UNROLL):
            base = (j * _UNROLL + u) * _LANES
            v = row_ref[pl.ds(base, _LANES)]
            vn = row_ref[pl.ds(base + 1, _LANES)]
            carry = step(v, vn, carry)
        return carry

    carry = lax.fori_loop(0, n_full, body,
                          (neg_inf, neg_inf, neg_inf, zero, zero))
    for j in range(n_full * _UNROLL, _VREGS - 1):
        v = row_ref[pl.ds(j * _LANES, _LANES)]
        vn = row_ref[pl.ds(j * _LANES + 1, _LANES)]
        carry = step(v, vn, carry)
    m1, m2, m3, acc, d2 = carry

    last = row_ref[pl.ds(_T - _LANES, _LANES)]
    acc = acc + last
    if with_top3:
        a = jnp.minimum(m1, last)
        m1 = jnp.maximum(m1, last)
        b = jnp.minimum(m2, a)
        m2 = jnp.maximum(m2, a)
        m3 = jnp.maximum(m3, b)
    s_d2 = _lane_total(d2)
    for l in range(_LANES - 1):
        d = last[l + 1] - last[l]
        s_d2 = s_d2 + d * d

    t3sum = jnp.float32(0.0)
    if with_top3:
        t1 = jnp.float32(-jnp.inf)
        t2 = jnp.float32(-jnp.inf)
        t3 = jnp.float32(-jnp.inf)
        for vec in (m1, m2, m3):
            for l in range(_LANES):
                v = vec[l]
                a = jnp.minimum(t1, v)
                t1 = jnp.maximum(t1, v)
                b = jnp.minimum(t2, a)
                t2 = jnp.maximum(t2, a)
                t3 = jnp.maximum(t3, b)
        t3sum = t1 + t2 + t3
    return t3sum, _lane_total(acc), s_d2


def _sc_main_body(anom_hbm, norm_hbm, out_hbm, row_v, vals_v):
    wid = lax.axis_index("s") * _NC + lax.axis_index("c")
    lane = lax.iota(jnp.int32, _LANES)

    vals = jnp.zeros((_LANES,), jnp.float32)
    a_sum = jnp.float32(0.0)
    a_d2 = jnp.float32(0.0)
    for i in range(_ROWS_PER_W):
        r = wid * _ROWS_PER_W + i
        pltpu.sync_copy(anom_hbm.at[r], row_v)
        s3, rs, rd2 = _row_stats(row_v, True)
        vals = jnp.where(lane == i, s3, vals)
        a_sum = a_sum + rs
        a_d2 = a_d2 + rd2

    n_sum = jnp.float32(0.0)
    n_d2 = jnp.float32(0.0)
    for i in range(_ROWS_PER_W):
        r = wid * _ROWS_PER_W + i
        pltpu.sync_copy(norm_hbm.at[r], row_v)
        _, rs, rd2 = _row_stats(row_v, False)
        n_sum = n_sum + rs
        n_d2 = n_d2 + rd2

    vals = jnp.where(lane == 4, a_sum, vals)
    vals = jnp.where(lane == 5, a_d2, vals)
    vals = jnp.where(lane == 6, n_sum, vals)
    vals = jnp.where(lane == 7, n_d2, vals)

    vals_v[...] = vals
    pltpu.sync_copy(vals_v, out_hbm.at[wid])


def _sc_main(anom, norm):
    mesh = plsc.VectorSubcoreMesh(core_axis_name="c", subcore_axis_name="s",
                                  num_cores=_NC, num_subcores=_NS)
    fn = pl.kernel(
        _sc_main_body,
        out_type=jax.ShapeDtypeStruct((_NC * _NS, _LANES), jnp.float32),
        mesh=mesh,
        scratch_types=[
            pltpu.VMEM((_T,), jnp.float32),
            pltpu.VMEM((_LANES,), jnp.float32),
        ],
    )
    return fn(anom, norm)


def _unkey(s):
    i = s ^ (lax.shift_right_arithmetic(s, 31) & jnp.int32(_I32_MAGN))
    return lax.bitcast_convert_type(i, jnp.float32)


def _topk_mean(x, k):
    rows = x.shape[0]
    kf = jnp.float32(k)

    def body(it, p):
        c = p | lax.shift_left(jnp.int32(1), 31 - it)
        tf = _unkey(c ^ jnp.int32(_I32_MIN))
        cnt = jnp.sum(jnp.where(x >= tf, 1.0, 0.0), axis=1, keepdims=True)
        return jnp.where(cnt >= kf, c, p)

    p = lax.fori_loop(0, 32, body, jnp.zeros((rows, 1), jnp.int32))
    tf = _unkey(p ^ jnp.int32(_I32_MIN))
    gt = x > tf
    n_gt = jnp.sum(jnp.where(gt, 1.0, 0.0), axis=1, keepdims=True)
    sum_gt = jnp.sum(jnp.where(gt, x, 0.0), axis=1, keepdims=True)
    return (sum_gt + (kf - n_gt) * tf) / kf


def _tc_main_body(norm_ref, nmean_ref):
    norm = norm_ref[...]
    t_n = norm.shape[1]
    hard_k = max(1, int(t_n * _HARD_NEG_RATIO))
    nmean_ref[...] = _topk_mean(norm, hard_k)


def _tc_main(norm):
    return pl.pallas_call(
        _tc_main_body,
        out_shape=jax.ShapeDtypeStruct((_B, 1), jnp.float32),
        in_specs=[pl.BlockSpec(memory_space=pltpu.VMEM)],
        out_specs=pl.BlockSpec(memory_space=pltpu.VMEM),
    )(norm)


def _tc_final_body(asums_ref, nmean_ref, total_ref, rank_ref, osp_ref,
                   osm_ref):
    asums = asums_ref[...]
    n_mean = nmean_ref[...]

    nw = _NC * _NS
    i_row = lax.broadcasted_iota(jnp.int32, (_B, nw), 0)
    w_col = lax.broadcasted_iota(jnp.int32, (_B, nw), 1)
    sel = jnp.where(w_col == lax.shift_right_logical(i_row, 2), 1.0, 0.0)
    b1 = lax.dot_general(sel, asums,
                         dimension_numbers=(((1,), (0,)), ((), ())),
                         preferred_element_type=jnp.float32)
    i_row2 = lax.broadcasted_iota(jnp.int32, (_B, _LANES), 0)
    c_col = lax.broadcasted_iota(jnp.int32, (_B, _LANES), 1)
    pick = jnp.where(c_col == (i_row2 & 3), 1.0, 0.0)
    a_sum3 = jnp.sum(b1 * pick, axis=1, keepdims=True)
    a_mean = a_sum3 / jnp.float32(_TOPK)

    lane_col = lax.broadcasted_iota(jnp.int32, (nw, _LANES), 1)

    def lane_sum(idx):
        return jnp.sum(jnp.where(lane_col == idx, asums, 0.0))

    sparsity = (lane_sum(4) + lane_sum(6)) * 0.5 / jnp.float32(_B * _T)
    smooth = (lane_sum(5) + lane_sum(7)) * 0.5 / jnp.float32(_B * (_T - 1))

    ones_col = jnp.ones((_B, 1), jnp.float32)
    norm_row = lax.dot_general(
        ones_col, n_mean,
        dimension_numbers=(((1,), (1,)), ((), ())),
        preferred_element_type=jnp.float32,
    )
    pairs = jnp.maximum(_MARGIN - a_mean + norm_row, 0.0)
    rank_loss = jnp.sum(pairs) / jnp.float32(_B * _B)

    total_ref[0, 0] = rank_loss + _LAMBDA_SPARSITY * sparsity \
        + _LAMBDA_SMOOTH * smooth
    rank_ref[0, 0] = rank_loss
    osp_ref[0, 0] = sparsity
    osm_ref[0, 0] = smooth


def _tc_final(a_sums, n_mean):
    scalar = jax.ShapeDtypeStruct((1, 1), jnp.float32)
    smem = pl.BlockSpec(memory_space=pltpu.SMEM)
    vmem = pl.BlockSpec(memory_space=pltpu.VMEM)
    return pl.pallas_call(
        _tc_final_body,
        out_shape=(scalar, scalar, scalar, scalar),
        in_specs=(vmem, vmem),
        out_specs=(smem, smem, smem, smem),
    )(a_sums, n_mean)


def kernel(anom_scores, norm_scores):
    sc_out = _sc_main(anom_scores, norm_scores)
    n_mean = _tc_main(norm_scores)
    total, rank, osp, osm = _tc_final(sc_out, n_mean)
    return (total[0, 0], rank[0, 0], osp[0, 0], osm[0, 0])
